# baseline (speedup 1.0000x reference)
import jax
import jax.numpy as jnp
import numpy as np
from jax import lax
from jax.experimental import pallas as pl

_PARAM_RAW_DIM = (4, 7, 9, 12, 16)
_N_TYPES = 5
_MAX_RAW = 16
_LATENT = 128
_EMB = 16
_PARAM_DIM = 32
_LANE = 128

_F32 = jnp.float32


def _fused_vae_kernel(params_ref, st_ref, eps_ref, mask_tbl_ref, W_pe_s_ref,
                      W_dr_s_ref, type_emb_ref,
                      b_pe_ref, W1_ref, b1_ref, W2_ref, b2_ref, W3_ref, b3_ref,
                      Wmu_ref, bmu_ref, Wlv_ref, blv_ref, Wc_ref, bc_ref,
                      Wd1_ref, bd1_ref, Wd2_ref, bd2_ref, Wd3_ref, bd3_ref,
                      b_dr_ref,
                      padded_ref, maskf_ref, logits_ref, mu_ref, lv_ref):
    bs = params_ref.shape[0]
    st = st_ref[:]
    oh = (st[:, None] == lax.broadcasted_iota(jnp.int32, (bs, _N_TYPES), 1)
          ).astype(_F32)
    st32 = jnp.broadcast_to(st[:, None], (bs, _PARAM_DIM))
    st16 = jnp.broadcast_to(st[:, None], (bs, _MAX_RAW))

    params = params_ref[:]

    pe_all = jnp.dot(params, W_pe_s_ref[:], preferred_element_type=_F32)
    pe = jnp.where(st32 == 0, pe_all[:, 0:_PARAM_DIM], 0.0)
    for t in range(1, _N_TYPES):
        pe += jnp.where(st32 == t,
                        pe_all[:, t * _LANE:t * _LANE + _PARAM_DIM], 0.0)

    W1a = W1_ref[0:_PARAM_DIM, :]
    W1b = W1_ref[_PARAM_DIM:_PARAM_DIM + _EMB, :]
    W1comb = (jnp.dot(b_pe_ref[:], W1a, preferred_element_type=_F32,
                      precision=lax.Precision.HIGHEST)
              + jnp.dot(type_emb_ref[:], W1b, preferred_element_type=_F32,
                        precision=lax.Precision.HIGHEST))
    h = (jnp.dot(pe, W1a, preferred_element_type=_F32)
         + jnp.dot(oh, W1comb, preferred_element_type=_F32,
                   precision=lax.Precision.HIGHEST))
    h = jnp.maximum(h + b1_ref[:][None, :], 0.0)
    h = jnp.dot(h, W2_ref[:], preferred_element_type=_F32)
    h = jnp.maximum(h + b2_ref[:][None, :], 0.0)
    h = (jnp.dot(h, W3_ref[:], preferred_element_type=_F32)
         + b3_ref[:][None, :])

    mu = jnp.dot(h, Wmu_ref[:], preferred_element_type=_F32) + bmu_ref[:][None, :]
    lv = jnp.dot(h, Wlv_ref[:], preferred_element_type=_F32) + blv_ref[:][None, :]
    mu_ref[:, :] = mu
    lv_ref[:, :] = lv

    std = jnp.exp(0.5 * jnp.clip(lv, -10.0, 10.0))
    z = mu + eps_ref[:] * std

    logits_ref[:, :] = (jnp.dot(z, Wc_ref[:], preferred_element_type=_F32)
                        + bc_ref[:][None, :])

    Wd1a = Wd1_ref[0:_LATENT, :]
    Wd1b = Wd1_ref[_LATENT:_LATENT + _EMB, :]
    Wd1comb = jnp.dot(type_emb_ref[:], Wd1b, preferred_element_type=_F32,
                      precision=lax.Precision.HIGHEST)
    hd = (jnp.dot(z, Wd1a, preferred_element_type=_F32)
          + jnp.dot(oh, Wd1comb, preferred_element_type=_F32,
                    precision=lax.Precision.HIGHEST))
    hd = jnp.maximum(hd + bd1_ref[:][None, :], 0.0)
    hd = jnp.dot(hd, Wd2_ref[:], preferred_element_type=_F32)
    hd = jnp.maximum(hd + bd2_ref[:][None, :], 0.0)
    pd = (jnp.dot(hd, Wd3_ref[:], preferred_element_type=_F32)
          + bd3_ref[:][None, :])

    out_all = jnp.dot(pd, W_dr_s_ref[:], preferred_element_type=_F32)
    out = jnp.dot(oh, b_dr_ref[:], preferred_element_type=_F32)
    for t in range(_N_TYPES):
        out += jnp.where(st16 == t,
                         out_all[:, t * _LANE:t * _LANE + _MAX_RAW], 0.0)
    padded_ref[:, :] = out

    maskf_ref[:, :] = jnp.dot(oh, mask_tbl_ref[:],
                              preferred_element_type=_F32) > 0.5


def _threefry2x32(k0, k1, x0, x1):
    R = (13, 15, 26, 6, 17, 29, 16, 24)
    ks = (np.uint32(k0), np.uint32(k1),
          np.uint32(np.uint32(k0) ^ np.uint32(k1) ^ np.uint32(0x1BD11BDA)))

    def rotl(x, d):
        return ((x << np.uint32(d)) | (x >> np.uint32(32 - d))).astype(np.uint32)

    x0 = (x0 + ks[0]).astype(np.uint32)
    x1 = (x1 + ks[1]).astype(np.uint32)
    for i in range(1, 6):
        for j in range(4):
            r = R[((i - 1) % 2) * 4 + j]
            x0 = (x0 + x1).astype(np.uint32)
            x1 = (rotl(x1, r) ^ x0).astype(np.uint32)
        x0 = (x0 + ks[i % 3]).astype(np.uint32)
        x1 = (x1 + ks[(i + 1) % 3] + np.uint32(i)).astype(np.uint32)
    return x0, x1


def _erfinv64(x):
    w = -np.log1p(-x * x)
    lt = w < 5.0
    wa = np.where(lt, w - 2.5, np.sqrt(np.maximum(w, 5.0)) - 3.0)
    ca = (2.81022636e-08, 3.43273939e-07, -3.5233877e-06, -4.39150654e-06,
          0.00021858087, -0.00125372503, -0.00417768164, 0.246640727,
          1.50140941)
    cb = (-0.000200214257, 0.000100950558, 0.00134934322, -0.00367342844,
          0.00573950773, -0.0076224613, 0.00943887047, 1.00167406, 2.83297682)
    pa = np.full_like(wa, ca[0])
    for c in ca[1:]:
        pa = c + pa * wa
    pb = np.full_like(wa, cb[0])
    for c in cb[1:]:
        pb = c + pb * wa
    return np.where(lt, pa, pb) * x


_EPS_CACHE = {}


def _eps_const(B):
    if B not in _EPS_CACHE:
        old = np.seterr(over="ignore")
        try:
            n = B * _LATENT
            idx = np.arange(n, dtype=np.uint64)
            x0 = (idx >> np.uint64(32)).astype(np.uint32)
            x1 = (idx & np.uint64(0xFFFFFFFF)).astype(np.uint32)
            o0, o1 = _threefry2x32(0, 42, x0, x1)
            bits = (o0 ^ o1).astype(np.uint32)
            float_bits = (bits >> np.uint32(9)) | np.uint32(0x3F800000)
            floats = float_bits.view(np.float32) - np.float32(1.0)
            lo = np.nextafter(np.float32(-1.0), np.float32(0.0),
                              dtype=np.float32)
            hi = np.float32(1.0)
            u = np.maximum(lo, (floats * (hi - lo) + lo).astype(np.float32))
            eps = (np.float32(np.sqrt(2.0))
                   * _erfinv64(u.astype(np.float64)).astype(np.float32))
            _EPS_CACHE[B] = eps.reshape(B, _LATENT)
        finally:
            np.seterr(**old)
    return _EPS_CACHE[B]


_MASK_TBL = (np.arange(_MAX_RAW)[None, :]
             < np.asarray(_PARAM_RAW_DIM)[:, None]).astype(np.float32)


def kernel(params, surface_type, type_emb, W_pe, b_pe, W1, b1, W2, b2, W3, b3,
           Wmu, bmu, Wlv, blv, Wc, bc, Wd1, bd1, Wd2, bd2, Wd3, bd3, W_dr, b_dr):
    B = params.shape[0]
    bs = 2048
    grid = (B // bs,)

    eps = _eps_const(B)
    st = surface_type.astype(jnp.int32)

    W_pe_s = jnp.pad(W_pe, ((0, 0), (0, 0), (0, _LANE - _PARAM_DIM)))
    W_pe_s = jnp.transpose(W_pe_s, (1, 0, 2)).reshape(_MAX_RAW,
                                                      _N_TYPES * _LANE)
    W_dr_s = jnp.pad(W_dr, ((0, 0), (0, 0), (0, _LANE - _MAX_RAW)))
    W_dr_s = jnp.transpose(W_dr_s, (1, 0, 2)).reshape(_PARAM_DIM,
                                                      _N_TYPES * _LANE)

    def row_spec(ncols):
        return pl.BlockSpec((bs, ncols), lambda i: (i, 0))

    def full_spec(arr):
        nd = arr.ndim
        return pl.BlockSpec(arr.shape, lambda i: (0,) * nd)

    weights = (jnp.asarray(_MASK_TBL), W_pe_s, W_dr_s, type_emb,
               b_pe, W1, b1, W2, b2, W3, b3, Wmu, bmu,
               Wlv, blv, Wc, bc, Wd1, bd1, Wd2, bd2, Wd3, bd3, b_dr)

    in_specs = ([row_spec(_MAX_RAW), pl.BlockSpec((bs,), lambda i: (i,)),
                 row_spec(_LATENT)] + [full_spec(w) for w in weights])

    out_shape = (
        jax.ShapeDtypeStruct((B, _MAX_RAW), _F32),
        jax.ShapeDtypeStruct((B, _MAX_RAW), jnp.bool_),
        jax.ShapeDtypeStruct((B, _N_TYPES), _F32),
        jax.ShapeDtypeStruct((B, _LATENT), _F32),
        jax.ShapeDtypeStruct((B, _LATENT), _F32),
    )
    out_specs = (row_spec(_MAX_RAW), row_spec(_MAX_RAW), row_spec(_N_TYPES),
                 row_spec(_LATENT), row_spec(_LATENT))

    padded, maskf, logits, mu, lv = pl.pallas_call(
        _fused_vae_kernel,
        grid=grid,
        in_specs=in_specs,
        out_specs=out_specs,
        out_shape=out_shape,
    )(params, st, eps, *weights)

    return (padded, maskf, logits, mu, lv)


# speedup vs baseline: 1.5749x; 1.5749x over previous
import jax
import jax.numpy as jnp
import numpy as np
from jax import lax
from jax.experimental import pallas as pl

_PARAM_RAW_DIM = (4, 7, 9, 12, 16)
_N_TYPES = 5
_MAX_RAW = 16
_LATENT = 128
_EMB = 16
_PARAM_DIM = 32
_LANE = 128

_F32 = jnp.float32


def _fused_vae_kernel(params_ref, st_ref, eps_ref, mask_tbl_ref, W_pe_s_ref,
                      W_dr_s_ref, type_emb_ref,
                      b_pe_ref, W1_ref, b1_ref, W2_ref, b2_ref, W3_ref, b3_ref,
                      Wmu_ref, bmu_ref, Wlv_ref, blv_ref, Wc_ref, bc_ref,
                      Wd1_ref, bd1_ref, Wd2_ref, bd2_ref, Wd3_ref, bd3_ref,
                      b_dr_ref,
                      padded_ref, maskf_ref, logits_ref, mu_ref, lv_ref):
    bs = params_ref.shape[0]
    st = st_ref[:]
    oh = (st[:, None] == lax.broadcasted_iota(jnp.int32, (bs, _N_TYPES), 1)
          ).astype(_F32)
    st32 = jnp.broadcast_to(st[:, None], (bs, _PARAM_DIM))
    st16 = jnp.broadcast_to(st[:, None], (bs, _MAX_RAW))

    params = params_ref[:]

    pe_all = jnp.dot(params, W_pe_s_ref[:], preferred_element_type=_F32)
    pe = jnp.where(st32 == 0, pe_all[:, 0:_PARAM_DIM], 0.0)
    for t in range(1, _N_TYPES):
        pe += jnp.where(st32 == t,
                        pe_all[:, t * _LANE:t * _LANE + _PARAM_DIM], 0.0)

    pe += jnp.dot(oh, b_pe_ref[:], preferred_element_type=_F32)
    emb = jnp.dot(oh, type_emb_ref[:], preferred_element_type=_F32)

    h = (jnp.dot(pe, W1_ref[0:_PARAM_DIM, :], preferred_element_type=_F32)
         + jnp.dot(emb, W1_ref[_PARAM_DIM:_PARAM_DIM + _EMB, :],
                   preferred_element_type=_F32))
    h = jnp.maximum(h + b1_ref[:][None, :], 0.0)
    h = jnp.dot(h, W2_ref[:], preferred_element_type=_F32)
    h = jnp.maximum(h + b2_ref[:][None, :], 0.0)
    h = (jnp.dot(h, W3_ref[:], preferred_element_type=_F32)
         + b3_ref[:][None, :])

    mu = jnp.dot(h, Wmu_ref[:], preferred_element_type=_F32) + bmu_ref[:][None, :]
    lv = jnp.dot(h, Wlv_ref[:], preferred_element_type=_F32) + blv_ref[:][None, :]
    mu_ref[:, :] = mu
    lv_ref[:, :] = lv

    std = jnp.exp(0.5 * jnp.clip(lv, -10.0, 10.0))
    z = mu + eps_ref[:] * std

    logits_ref[:, :] = (jnp.dot(z, Wc_ref[:], preferred_element_type=_F32)
                        + bc_ref[:][None, :])

    hd = (jnp.dot(z, Wd1_ref[0:_LATENT, :], preferred_element_type=_F32)
          + jnp.dot(emb, Wd1_ref[_LATENT:_LATENT + _EMB, :],
                    preferred_element_type=_F32))
    hd = jnp.maximum(hd + bd1_ref[:][None, :], 0.0)
    hd = jnp.dot(hd, Wd2_ref[:], preferred_element_type=_F32)
    hd = jnp.maximum(hd + bd2_ref[:][None, :], 0.0)
    pd = (jnp.dot(hd, Wd3_ref[:], preferred_element_type=_F32)
          + bd3_ref[:][None, :])

    out_all = jnp.dot(pd, W_dr_s_ref[:], preferred_element_type=_F32)
    out = jnp.dot(oh, b_dr_ref[:], preferred_element_type=_F32)
    for t in range(_N_TYPES):
        out += jnp.where(st16 == t,
                         out_all[:, t * _LANE:t * _LANE + _MAX_RAW], 0.0)
    padded_ref[:, :] = out

    maskf_ref[:, :] = jnp.dot(oh, mask_tbl_ref[:],
                              preferred_element_type=_F32) > 0.5


def _threefry2x32(k0, k1, x0, x1):
    R = (13, 15, 26, 6, 17, 29, 16, 24)
    ks = (np.uint32(k0), np.uint32(k1),
          np.uint32(np.uint32(k0) ^ np.uint32(k1) ^ np.uint32(0x1BD11BDA)))

    def rotl(x, d):
        return ((x << np.uint32(d)) | (x >> np.uint32(32 - d))).astype(np.uint32)

    x0 = (x0 + ks[0]).astype(np.uint32)
    x1 = (x1 + ks[1]).astype(np.uint32)
    for i in range(1, 6):
        for j in range(4):
            r = R[((i - 1) % 2) * 4 + j]
            x0 = (x0 + x1).astype(np.uint32)
            x1 = (rotl(x1, r) ^ x0).astype(np.uint32)
        x0 = (x0 + ks[i % 3]).astype(np.uint32)
        x1 = (x1 + ks[(i + 1) % 3] + np.uint32(i)).astype(np.uint32)
    return x0, x1


def _erfinv64(x):
    w = -np.log1p(-x * x)
    lt = w < 5.0
    wa = np.where(lt, w - 2.5, np.sqrt(np.maximum(w, 5.0)) - 3.0)
    ca = (2.81022636e-08, 3.43273939e-07, -3.5233877e-06, -4.39150654e-06,
          0.00021858087, -0.00125372503, -0.00417768164, 0.246640727,
          1.50140941)
    cb = (-0.000200214257, 0.000100950558, 0.00134934322, -0.00367342844,
          0.00573950773, -0.0076224613, 0.00943887047, 1.00167406, 2.83297682)
    pa = np.full_like(wa, ca[0])
    for c in ca[1:]:
        pa = c + pa * wa
    pb = np.full_like(wa, cb[0])
    for c in cb[1:]:
        pb = c + pb * wa
    return np.where(lt, pa, pb) * x


_EPS_CACHE = {}


def _eps_const(B):
    if B not in _EPS_CACHE:
        old = np.seterr(over="ignore")
        try:
            n = B * _LATENT
            idx = np.arange(n, dtype=np.uint64)
            x0 = (idx >> np.uint64(32)).astype(np.uint32)
            x1 = (idx & np.uint64(0xFFFFFFFF)).astype(np.uint32)
            o0, o1 = _threefry2x32(0, 42, x0, x1)
            bits = (o0 ^ o1).astype(np.uint32)
            float_bits = (bits >> np.uint32(9)) | np.uint32(0x3F800000)
            floats = float_bits.view(np.float32) - np.float32(1.0)
            lo = np.nextafter(np.float32(-1.0), np.float32(0.0),
                              dtype=np.float32)
            hi = np.float32(1.0)
            u = np.maximum(lo, (floats * (hi - lo) + lo).astype(np.float32))
            eps = (np.float32(np.sqrt(2.0))
                   * _erfinv64(u.astype(np.float64)).astype(np.float32))
            _EPS_CACHE[B] = eps.reshape(B, _LATENT)
        finally:
            np.seterr(**old)
    return _EPS_CACHE[B]


_MASK_TBL = (np.arange(_MAX_RAW)[None, :]
             < np.asarray(_PARAM_RAW_DIM)[:, None]).astype(np.float32)


def kernel(params, surface_type, type_emb, W_pe, b_pe, W1, b1, W2, b2, W3, b3,
           Wmu, bmu, Wlv, blv, Wc, bc, Wd1, bd1, Wd2, bd2, Wd3, bd3, W_dr, b_dr):
    B = params.shape[0]
    bs = 2048
    grid = (B // bs,)

    eps = _eps_const(B)
    st = surface_type.astype(jnp.int32)

    W_pe_s = jnp.pad(W_pe, ((0, 0), (0, 0), (0, _LANE - _PARAM_DIM)))
    W_pe_s = jnp.transpose(W_pe_s, (1, 0, 2)).reshape(_MAX_RAW,
                                                      _N_TYPES * _LANE)
    W_dr_s = jnp.pad(W_dr, ((0, 0), (0, 0), (0, _LANE - _MAX_RAW)))
    W_dr_s = jnp.transpose(W_dr_s, (1, 0, 2)).reshape(_PARAM_DIM,
                                                      _N_TYPES * _LANE)

    def row_spec(ncols):
        return pl.BlockSpec((bs, ncols), lambda i: (i, 0))

    def full_spec(arr):
        nd = arr.ndim
        return pl.BlockSpec(arr.shape, lambda i: (0,) * nd)

    weights = (jnp.asarray(_MASK_TBL), W_pe_s, W_dr_s, type_emb,
               b_pe, W1, b1, W2, b2, W3, b3, Wmu, bmu,
               Wlv, blv, Wc, bc, Wd1, bd1, Wd2, bd2, Wd3, bd3, b_dr)

    in_specs = ([row_spec(_MAX_RAW), pl.BlockSpec((bs,), lambda i: (i,)),
                 row_spec(_LATENT)] + [full_spec(w) for w in weights])

    out_shape = (
        jax.ShapeDtypeStruct((B, _MAX_RAW), _F32),
        jax.ShapeDtypeStruct((B, _MAX_RAW), jnp.bool_),
        jax.ShapeDtypeStruct((B, _N_TYPES), _F32),
        jax.ShapeDtypeStruct((B, _LATENT), _F32),
        jax.ShapeDtypeStruct((B, _LATENT), _F32),
    )
    out_specs = (row_spec(_MAX_RAW), row_spec(_MAX_RAW), row_spec(_N_TYPES),
                 row_spec(_LATENT), row_spec(_LATENT))

    padded, maskf, logits, mu, lv = pl.pallas_call(
        _fused_vae_kernel,
        grid=grid,
        in_specs=in_specs,
        out_specs=out_specs,
        out_shape=out_shape,
    )(params, st, eps, *weights)

    return (padded, maskf, logits, mu, lv)


# speedup vs baseline: 2.0604x; 1.3083x over previous
import jax
import jax.numpy as jnp
import numpy as np
from jax import lax
from jax.experimental import pallas as pl

_PARAM_RAW_DIM = (4, 7, 9, 12, 16)
_N_TYPES = 5
_MAX_RAW = 16
_LATENT = 128
_EMB = 16
_PARAM_DIM = 32
_LANE = 128

_F32 = jnp.float32


def _fused_vae_kernel(params_ref, st_ref, eps_ref, mask_tbl_ref, W_pe_s_ref,
                      W_dr_s_ref, type_emb_ref,
                      b_pe_ref, W1_ref, b1_ref, W2_ref, b2_ref, W3_ref, b3_ref,
                      Wmu_ref, bmu_ref, Wlv_ref, blv_ref, Wc_ref, bc_ref,
                      Wd1_ref, bd1_ref, Wd2_ref, bd2_ref, Wd3_ref, bd3_ref,
                      b_dr_ref,
                      padded_ref, maski_ref, logits_ref, mu_ref, lv_ref):
    bs = params_ref.shape[1]
    st = st_ref[:]
    oh = (st[:, None] == lax.broadcasted_iota(jnp.int32, (bs, _N_TYPES), 1)
          ).astype(_F32)
    st32 = jnp.broadcast_to(st[:, None], (bs, _PARAM_DIM))
    st16 = jnp.broadcast_to(st[:, None], (bs, _MAX_RAW))

    pe_all = lax.dot_general(params_ref[:], W_pe_s_ref[:],
                             (((0,), (0,)), ((), ())),
                             preferred_element_type=_F32)
    pe = jnp.where(st32 == 0, pe_all[:, 0:_PARAM_DIM], 0.0)
    for t in range(1, _N_TYPES):
        pe += jnp.where(st32 == t,
                        pe_all[:, t * _LANE:t * _LANE + _PARAM_DIM], 0.0)

    pe += jnp.dot(oh, b_pe_ref[:], preferred_element_type=_F32)
    emb = jnp.dot(oh, type_emb_ref[:], preferred_element_type=_F32)

    h = (jnp.dot(pe, W1_ref[0:_PARAM_DIM, :], preferred_element_type=_F32)
         + jnp.dot(emb, W1_ref[_PARAM_DIM:_PARAM_DIM + _EMB, :],
                   preferred_element_type=_F32))
    h = jnp.maximum(h + b1_ref[:][None, :], 0.0)
    h = jnp.dot(h, W2_ref[:], preferred_element_type=_F32)
    h = jnp.maximum(h + b2_ref[:][None, :], 0.0)
    h = (jnp.dot(h, W3_ref[:], preferred_element_type=_F32)
         + b3_ref[:][None, :])

    mu = jnp.dot(h, Wmu_ref[:], preferred_element_type=_F32) + bmu_ref[:][None, :]
    lv = jnp.dot(h, Wlv_ref[:], preferred_element_type=_F32) + blv_ref[:][None, :]
    mu_ref[:, :] = mu
    lv_ref[:, :] = lv

    std = jnp.exp(0.5 * jnp.clip(lv, -10.0, 10.0))
    z = mu + eps_ref[:] * std

    logits_ref[:, :] = (jnp.dot(z, Wc_ref[:], preferred_element_type=_F32)
                        + bc_ref[:][None, :]).T

    hd = (jnp.dot(z, Wd1_ref[0:_LATENT, :], preferred_element_type=_F32)
          + jnp.dot(emb, Wd1_ref[_LATENT:_LATENT + _EMB, :],
                    preferred_element_type=_F32))
    hd = jnp.maximum(hd + bd1_ref[:][None, :], 0.0)
    hd = jnp.dot(hd, Wd2_ref[:], preferred_element_type=_F32)
    hd = jnp.maximum(hd + bd2_ref[:][None, :], 0.0)
    pd = (jnp.dot(hd, Wd3_ref[:], preferred_element_type=_F32)
          + bd3_ref[:][None, :])

    out_all = jnp.dot(pd, W_dr_s_ref[:], preferred_element_type=_F32)
    out = jnp.dot(oh, b_dr_ref[:], preferred_element_type=_F32)
    for t in range(_N_TYPES):
        out += jnp.where(st16 == t,
                         out_all[:, t * _LANE:t * _LANE + _MAX_RAW], 0.0)
    padded_ref[:, :] = out.T

    maskf = jnp.dot(oh, mask_tbl_ref[:], preferred_element_type=_F32)
    maski_ref[:, :] = (maskf > 0.5).astype(jnp.int32).T


def _threefry2x32(k0, k1, x0, x1):
    R = (13, 15, 26, 6, 17, 29, 16, 24)
    ks = (np.uint32(k0), np.uint32(k1),
          np.uint32(np.uint32(k0) ^ np.uint32(k1) ^ np.uint32(0x1BD11BDA)))

    def rotl(x, d):
        return ((x << np.uint32(d)) | (x >> np.uint32(32 - d))).astype(np.uint32)

    x0 = (x0 + ks[0]).astype(np.uint32)
    x1 = (x1 + ks[1]).astype(np.uint32)
    for i in range(1, 6):
        for j in range(4):
            r = R[((i - 1) % 2) * 4 + j]
            x0 = (x0 + x1).astype(np.uint32)
            x1 = (rotl(x1, r) ^ x0).astype(np.uint32)
        x0 = (x0 + ks[i % 3]).astype(np.uint32)
        x1 = (x1 + ks[(i + 1) % 3] + np.uint32(i)).astype(np.uint32)
    return x0, x1


def _erfinv64(x):
    w = -np.log1p(-x * x)
    lt = w < 5.0
    wa = np.where(lt, w - 2.5, np.sqrt(np.maximum(w, 5.0)) - 3.0)
    ca = (2.81022636e-08, 3.43273939e-07, -3.5233877e-06, -4.39150654e-06,
          0.00021858087, -0.00125372503, -0.00417768164, 0.246640727,
          1.50140941)
    cb = (-0.000200214257, 0.000100950558, 0.00134934322, -0.00367342844,
          0.00573950773, -0.0076224613, 0.00943887047, 1.00167406, 2.83297682)
    pa = np.full_like(wa, ca[0])
    for c in ca[1:]:
        pa = c + pa * wa
    pb = np.full_like(wa, cb[0])
    for c in cb[1:]:
        pb = c + pb * wa
    return np.where(lt, pa, pb) * x


_EPS_CACHE = {}


def _eps_const(B):
    if B not in _EPS_CACHE:
        old = np.seterr(over="ignore")
        try:
            n = B * _LATENT
            idx = np.arange(n, dtype=np.uint64)
            x0 = (idx >> np.uint64(32)).astype(np.uint32)
            x1 = (idx & np.uint64(0xFFFFFFFF)).astype(np.uint32)
            o0, o1 = _threefry2x32(0, 42, x0, x1)
            bits = (o0 ^ o1).astype(np.uint32)
            float_bits = (bits >> np.uint32(9)) | np.uint32(0x3F800000)
            floats = float_bits.view(np.float32) - np.float32(1.0)
            lo = np.nextafter(np.float32(-1.0), np.float32(0.0),
                              dtype=np.float32)
            hi = np.float32(1.0)
            u = np.maximum(lo, (floats * (hi - lo) + lo).astype(np.float32))
            eps = (np.float32(np.sqrt(2.0))
                   * _erfinv64(u.astype(np.float64)).astype(np.float32))
            _EPS_CACHE[B] = eps.reshape(B, _LATENT)
        finally:
            np.seterr(**old)
    return _EPS_CACHE[B]


_MASK_TBL = (np.arange(_MAX_RAW)[None, :]
             < np.asarray(_PARAM_RAW_DIM)[:, None]).astype(np.float32)


def kernel(params, surface_type, type_emb, W_pe, b_pe, W1, b1, W2, b2, W3, b3,
           Wmu, bmu, Wlv, blv, Wc, bc, Wd1, bd1, Wd2, bd2, Wd3, bd3, W_dr, b_dr):
    B = params.shape[0]
    bs = 2048
    grid = (B // bs,)

    eps = _eps_const(B)
    st = surface_type.astype(jnp.int32)

    W_pe_s = jnp.pad(W_pe, ((0, 0), (0, 0), (0, _LANE - _PARAM_DIM)))
    W_pe_s = jnp.transpose(W_pe_s, (1, 0, 2)).reshape(_MAX_RAW,
                                                      _N_TYPES * _LANE)
    W_dr_s = jnp.pad(W_dr, ((0, 0), (0, 0), (0, _LANE - _MAX_RAW)))
    W_dr_s = jnp.transpose(W_dr_s, (1, 0, 2)).reshape(_PARAM_DIM,
                                                      _N_TYPES * _LANE)

    def row_spec(ncols):
        return pl.BlockSpec((bs, ncols), lambda i: (i, 0))

    def col_spec(nrows):
        return pl.BlockSpec((nrows, bs), lambda i: (0, i))

    def full_spec(arr):
        nd = arr.ndim
        return pl.BlockSpec(arr.shape, lambda i: (0,) * nd)

    weights = (jnp.asarray(_MASK_TBL), W_pe_s, W_dr_s, type_emb,
               b_pe, W1, b1, W2, b2, W3, b3, Wmu, bmu,
               Wlv, blv, Wc, bc, Wd1, bd1, Wd2, bd2, Wd3, bd3, b_dr)

    in_specs = ([col_spec(_MAX_RAW), pl.BlockSpec((bs,), lambda i: (i,)),
                 row_spec(_LATENT)] + [full_spec(w) for w in weights])

    out_shape = (
        jax.ShapeDtypeStruct((_MAX_RAW, B), _F32),
        jax.ShapeDtypeStruct((_MAX_RAW, B), jnp.int32),
        jax.ShapeDtypeStruct((_N_TYPES, B), _F32),
        jax.ShapeDtypeStruct((B, _LATENT), _F32),
        jax.ShapeDtypeStruct((B, _LATENT), _F32),
    )
    out_specs = (col_spec(_MAX_RAW), col_spec(_MAX_RAW), col_spec(_N_TYPES),
                 row_spec(_LATENT), row_spec(_LATENT))

    padded_t, maski_t, logits_t, mu, lv = pl.pallas_call(
        _fused_vae_kernel,
        grid=grid,
        in_specs=in_specs,
        out_specs=out_specs,
        out_shape=out_shape,
    )(params.T, st, eps, *weights)

    return (padded_t.T, maski_t.T.astype(jnp.bool_), logits_t.T, mu, lv)


# speedup vs baseline: 2.1806x; 1.0583x over previous
import jax
import jax.numpy as jnp
import numpy as np
from jax import lax
from jax.experimental import pallas as pl

_PARAM_RAW_DIM = (4, 7, 9, 12, 16)
_N_TYPES = 5
_MAX_RAW = 16
_LATENT = 128
_EMB = 16
_PARAM_DIM = 32
_LANE = 128

_F32 = jnp.float32


def _fused_vae_kernel(params_ref, st_ref, eps_ref, mask_tbl_ref, Wx_s_ref,
                      type_emb_ref,
                      b_pe_ref, W1_ref, b1_ref, W2_ref, b2_ref, W3_ref, b3_ref,
                      Wmu_ref, bmu_ref, Wlv_ref, blv_ref, WcT_ref, bc_ref,
                      Wd1_ref, bd1_ref, Wd2_ref, bd2_ref, Wd3T_ref, bd3_ref,
                      b_dr_ref,
                      padded_ref, maski_ref, logits_ref, mu_ref, lv_ref):
    bs = params_ref.shape[1]
    st = st_ref[:]
    oh = (st[:, None] == lax.broadcasted_iota(jnp.int32, (bs, _N_TYPES), 1)
          ).astype(_F32)
    st32 = jnp.broadcast_to(st[:, None], (bs, _PARAM_DIM))
    st16 = jnp.broadcast_to(st[:, None], (bs, _MAX_RAW))

    pe_all = lax.dot_general(params_ref[:], Wx_s_ref[0:_MAX_RAW, :],
                             (((0,), (0,)), ((), ())),
                             preferred_element_type=_F32)
    pe = jnp.where(st32 == 0, pe_all[:, 0:_PARAM_DIM], 0.0)
    for t in range(1, _N_TYPES):
        pe += jnp.where(st32 == t,
                        pe_all[:, t * _LANE:t * _LANE + _PARAM_DIM], 0.0)

    pe += jnp.dot(oh, b_pe_ref[:], preferred_element_type=_F32)
    emb = jnp.dot(oh, type_emb_ref[:], preferred_element_type=_F32)

    h = (jnp.dot(pe, W1_ref[0:_PARAM_DIM, :], preferred_element_type=_F32)
         + jnp.dot(emb, W1_ref[_PARAM_DIM:_PARAM_DIM + _EMB, :],
                   preferred_element_type=_F32))
    h = jnp.maximum(h + b1_ref[:][None, :], 0.0)
    h = jnp.dot(h, W2_ref[:], preferred_element_type=_F32)
    h = jnp.maximum(h + b2_ref[:][None, :], 0.0)
    h = (jnp.dot(h, W3_ref[:], preferred_element_type=_F32)
         + b3_ref[:][None, :])

    mu = jnp.dot(h, Wmu_ref[:], preferred_element_type=_F32) + bmu_ref[:][None, :]
    lv = jnp.dot(h, Wlv_ref[:], preferred_element_type=_F32) + blv_ref[:][None, :]
    mu_ref[:, :] = mu
    lv_ref[:, :] = lv

    std = jnp.exp(0.5 * jnp.clip(lv, -10.0, 10.0))
    z = mu + eps_ref[:] * std

    logits_ref[:, :] = (lax.dot_general(z, WcT_ref[:],
                                        (((1,), (1,)), ((), ())),
                                        preferred_element_type=_F32)
                        + bc_ref[:][None, :]).T

    hd = (jnp.dot(z, Wd1_ref[0:_LATENT, :], preferred_element_type=_F32)
          + jnp.dot(emb, Wd1_ref[_LATENT:_LATENT + _EMB, :],
                    preferred_element_type=_F32))
    hd = jnp.maximum(hd + bd1_ref[:][None, :], 0.0)
    hd = jnp.dot(hd, Wd2_ref[:], preferred_element_type=_F32)
    hd = jnp.maximum(hd + bd2_ref[:][None, :], 0.0)
    pd = (lax.dot_general(hd, Wd3T_ref[:], (((1,), (1,)), ((), ())),
                          preferred_element_type=_F32)
          + bd3_ref[:][None, :])

    out_all = jnp.dot(pd, Wx_s_ref[_MAX_RAW:_MAX_RAW + _PARAM_DIM, :],
                      preferred_element_type=_F32)
    out = jnp.dot(oh, b_dr_ref[:], preferred_element_type=_F32)
    for t in range(_N_TYPES):
        out += jnp.where(st16 == t,
                         out_all[:, t * _LANE:t * _LANE + _MAX_RAW], 0.0)
    padded_ref[:, :] = out.T

    maskf = jnp.dot(oh, mask_tbl_ref[:], preferred_element_type=_F32)
    maski_ref[:, :] = (maskf > 0.5).astype(jnp.int32).T


def _threefry2x32(k0, k1, x0, x1):
    R = (13, 15, 26, 6, 17, 29, 16, 24)
    ks = (np.uint32(k0), np.uint32(k1),
          np.uint32(np.uint32(k0) ^ np.uint32(k1) ^ np.uint32(0x1BD11BDA)))

    def rotl(x, d):
        return ((x << np.uint32(d)) | (x >> np.uint32(32 - d))).astype(np.uint32)

    x0 = (x0 + ks[0]).astype(np.uint32)
    x1 = (x1 + ks[1]).astype(np.uint32)
    for i in range(1, 6):
        for j in range(4):
            r = R[((i - 1) % 2) * 4 + j]
            x0 = (x0 + x1).astype(np.uint32)
            x1 = (rotl(x1, r) ^ x0).astype(np.uint32)
        x0 = (x0 + ks[i % 3]).astype(np.uint32)
        x1 = (x1 + ks[(i + 1) % 3] + np.uint32(i)).astype(np.uint32)
    return x0, x1


def _erfinv64(x):
    w = -np.log1p(-x * x)
    lt = w < 5.0
    wa = np.where(lt, w - 2.5, np.sqrt(np.maximum(w, 5.0)) - 3.0)
    ca = (2.81022636e-08, 3.43273939e-07, -3.5233877e-06, -4.39150654e-06,
          0.00021858087, -0.00125372503, -0.00417768164, 0.246640727,
          1.50140941)
    cb = (-0.000200214257, 0.000100950558, 0.00134934322, -0.00367342844,
          0.00573950773, -0.0076224613, 0.00943887047, 1.00167406, 2.83297682)
    pa = np.full_like(wa, ca[0])
    for c in ca[1:]:
        pa = c + pa * wa
    pb = np.full_like(wa, cb[0])
    for c in cb[1:]:
        pb = c + pb * wa
    return np.where(lt, pa, pb) * x


_EPS_CACHE = {}


def _eps_const(B):
    if B not in _EPS_CACHE:
        old = np.seterr(over="ignore")
        try:
            n = B * _LATENT
            idx = np.arange(n, dtype=np.uint64)
            x0 = (idx >> np.uint64(32)).astype(np.uint32)
            x1 = (idx & np.uint64(0xFFFFFFFF)).astype(np.uint32)
            o0, o1 = _threefry2x32(0, 42, x0, x1)
            bits = (o0 ^ o1).astype(np.uint32)
            float_bits = (bits >> np.uint32(9)) | np.uint32(0x3F800000)
            floats = float_bits.view(np.float32) - np.float32(1.0)
            lo = np.nextafter(np.float32(-1.0), np.float32(0.0),
                              dtype=np.float32)
            hi = np.float32(1.0)
            u = np.maximum(lo, (floats * (hi - lo) + lo).astype(np.float32))
            eps = (np.float32(np.sqrt(2.0))
                   * _erfinv64(u.astype(np.float64)).astype(np.float32))
            _EPS_CACHE[B] = eps.reshape(B, _LATENT)
        finally:
            np.seterr(**old)
    return _EPS_CACHE[B]


_MASK_TBL = (np.arange(_MAX_RAW)[None, :]
             < np.asarray(_PARAM_RAW_DIM)[:, None]).astype(np.float32)


def kernel(params, surface_type, type_emb, W_pe, b_pe, W1, b1, W2, b2, W3, b3,
           Wmu, bmu, Wlv, blv, Wc, bc, Wd1, bd1, Wd2, bd2, Wd3, bd3, W_dr, b_dr):
    B = params.shape[0]
    bs = 2048
    grid = (B // bs,)

    eps = _eps_const(B)
    st = surface_type.astype(jnp.int32)

    W_pe_s = jnp.transpose(
        jnp.pad(W_pe, ((0, 0), (0, 0), (0, _LANE - _PARAM_DIM))),
        (1, 0, 2)).reshape(_MAX_RAW, _N_TYPES * _LANE)
    W_dr_s = jnp.transpose(
        jnp.pad(W_dr, ((0, 0), (0, 0), (0, _LANE - _MAX_RAW))),
        (1, 0, 2)).reshape(_PARAM_DIM, _N_TYPES * _LANE)
    Wx_s = jnp.concatenate([W_pe_s, W_dr_s], axis=0)

    def row_spec(ncols):
        return pl.BlockSpec((bs, ncols), lambda i: (i, 0))

    def col_spec(nrows):
        return pl.BlockSpec((nrows, bs), lambda i: (0, i))

    def full_spec(arr):
        nd = arr.ndim
        return pl.BlockSpec(arr.shape, lambda i: (0,) * nd)

    weights = (jnp.asarray(_MASK_TBL), Wx_s, type_emb,
               b_pe, W1, b1, W2, b2, W3, b3, Wmu, bmu,
               Wlv, blv, Wc.T, bc, Wd1, bd1, Wd2, bd2, Wd3.T, bd3, b_dr)

    in_specs = ([col_spec(_MAX_RAW), pl.BlockSpec((bs,), lambda i: (i,)),
                 row_spec(_LATENT)] + [full_spec(w) for w in weights])

    out_shape = (
        jax.ShapeDtypeStruct((_MAX_RAW, B), _F32),
        jax.ShapeDtypeStruct((_MAX_RAW, B), jnp.int32),
        jax.ShapeDtypeStruct((_N_TYPES, B), _F32),
        jax.ShapeDtypeStruct((B, _LATENT), _F32),
        jax.ShapeDtypeStruct((B, _LATENT), _F32),
    )
    out_specs = (col_spec(_MAX_RAW), col_spec(_MAX_RAW), col_spec(_N_TYPES),
                 row_spec(_LATENT), row_spec(_LATENT))

    padded_t, maski_t, logits_t, mu, lv = pl.pallas_call(
        _fused_vae_kernel,
        grid=grid,
        in_specs=in_specs,
        out_specs=out_specs,
        out_shape=out_shape,
    )(params.T, st, eps, *weights)

    return (padded_t.T, maski_t.T.astype(jnp.bool_), logits_t.T, mu, lv)
